# baseline (speedup 1.0000x reference)
import jax
import jax.numpy as jnp
from jax import lax
from jax.experimental import pallas as pl
from jax.experimental.pallas import tpu as pltpu
from jax.experimental.pallas import tpu_sc as plsc

BATCH = 16384
HIDDEN = 64
NUM_CORES = 2
NUM_SUBCORES = 16
NUM_WORKERS = NUM_CORES * NUM_SUBCORES
B_PER_W = BATCH // NUM_WORKERS
TCOL = 128
NRING = 11
N_STEPS = B_PER_W // NRING
N_REM = B_PER_W - N_STEPS * NRING


def _embed_body(tT, labels_hbm, outT, lab_v, out_vT,
                b0, b1, b2, b3, b4, b5, b6, b7, b8, b9, b10,
                s0, s1, s2, s3, s4, s5, s6, s7, s8, s9, s10):
    wid = lax.axis_index("s") * NUM_CORES + lax.axis_index("c")
    base = pl.multiple_of(wid * B_PER_W, TCOL)
    pltpu.sync_copy(labels_hbm.at[pl.ds(base, B_PER_W)], lab_v)

    bufs = (b0, b1, b2, b3, b4, b5, b6, b7, b8, b9, b10)
    sems = (s0, s1, s2, s3, s4, s5, s6, s7, s8, s9, s10)
    iota = lax.iota(jnp.int32, 16)

    def get_label(i):
        gb = pl.multiple_of((i >> 4) * 16, 16)
        lv = lab_v[pl.ds(gb, 16)]
        return jnp.max(jnp.where(iota == (i & 15), lv, 0))

    def enq(i, k):
        r = get_label(i)
        rb = pl.multiple_of((r >> 7) * TCOL, TCOL)
        for a in range(8):
            pltpu.async_copy(tT.at[pl.ds(a * 8, 8), pl.ds(rb, TCOL)],
                             bufs[k].at[pl.ds(a * 8, 8)], sems[k])

    def extract(i, k):
        for a in range(8):
            pltpu.make_async_copy(tT.at[pl.ds(0, 8), pl.ds(0, TCOL)],
                                  bufs[k].at[pl.ds(0, 8)], sems[k]).wait()
        r = get_label(i)
        col = jnp.full((16,), r & (TCOL - 1), jnp.int32)
        tv = jnp.full((16,), i, jnp.int32)
        for q in range(HIDDEN // 16):
            cv = iota + q * 16
            vals = plsc.load_gather(bufs[k], [cv, col])
            plsc.store_scatter(out_vT, [cv, tv], vals)

    for k in range(NRING):
        enq(k, k)

    def steady(s, carry):
        for k in range(NRING):
            extract((s - 1) * NRING + k, k)
            enq(s * NRING + k, k)
        return carry

    lax.fori_loop(1, N_STEPS, steady, 0)
    for k in range(NRING):
        extract((N_STEPS - 1) * NRING + k, k)
    for j in range(N_REM):
        enq(N_STEPS * NRING + j, j)
    for j in range(N_REM):
        extract(N_STEPS * NRING + j, j)

    pltpu.sync_copy(out_vT, outT.at[:, pl.ds(base, B_PER_W)])


def kernel(labels, embedding_table):
    tT = embedding_table.T
    mesh = plsc.VectorSubcoreMesh(core_axis_name="c", subcore_axis_name="s")
    run = pl.kernel(
        _embed_body,
        mesh=mesh,
        out_type=jax.ShapeDtypeStruct((HIDDEN, BATCH), jnp.float32),
        scratch_types=[
            pltpu.VMEM((B_PER_W,), jnp.int32),
            pltpu.VMEM((HIDDEN, B_PER_W), jnp.float32),
        ] + [pltpu.VMEM((HIDDEN, TCOL), jnp.float32)] * NRING
          + [pltpu.SemaphoreType.DMA] * NRING,
        compiler_params=pltpu.CompilerParams(
            needs_layout_passes=False, disable_bounds_checks=True),
    )
    outT = run(tT, labels)
    return outT.T


# speedup vs baseline: 1.0139x; 1.0139x over previous
import jax
import jax.numpy as jnp
from jax import lax
from jax.experimental import pallas as pl
from jax.experimental.pallas import tpu as pltpu
from jax.experimental.pallas import tpu_sc as plsc

BATCH = 16384
HIDDEN = 64
NUM_CORES = 2
NUM_SUBCORES = 16
NUM_WORKERS = NUM_CORES * NUM_SUBCORES
B_PER_W = BATCH // NUM_WORKERS
TCOL = 128
NRING = 11
N_STEPS = B_PER_W // NRING
N_REM = B_PER_W - N_STEPS * NRING


def _embed_body(tT, labels_hbm, outT, lab_v, out_vT,
                b0, b1, b2, b3, b4, b5, b6, b7, b8, b9, b10,
                s0, s1, s2, s3, s4, s5, s6, s7, s8, s9, s10):
    wid = lax.axis_index("s") * NUM_CORES + lax.axis_index("c")
    base = pl.multiple_of(wid * B_PER_W, TCOL)
    pltpu.sync_copy(labels_hbm.at[pl.ds(base, B_PER_W)], lab_v)

    bufs = (b0, b1, b2, b3, b4, b5, b6, b7, b8, b9, b10)
    sems = (s0, s1, s2, s3, s4, s5, s6, s7, s8, s9, s10)
    iota = lax.iota(jnp.int32, 16)

    def get_label(i):
        gb = pl.multiple_of((i >> 4) * 16, 16)
        lv = lab_v[pl.ds(gb, 16)]
        return jnp.max(jnp.where(iota == (i & 15), lv, 0))

    def enq(i, k):
        r = get_label(i)
        rb = pl.multiple_of((r >> 7) * TCOL, TCOL)
        pltpu.async_copy(tT.at[:, pl.ds(rb, TCOL)], bufs[k], sems[k])

    def extract(i, k):
        pltpu.make_async_copy(tT.at[:, pl.ds(0, TCOL)], bufs[k],
                              sems[k]).wait()
        r = get_label(i)
        col = jnp.full((16,), r & (TCOL - 1), jnp.int32)
        tv = jnp.full((16,), i, jnp.int32)
        for q in range(HIDDEN // 16):
            cv = iota + q * 16
            vals = plsc.load_gather(bufs[k], [cv, col])
            plsc.store_scatter(out_vT, [cv, tv], vals)

    for k in range(NRING):
        enq(k, k)

    def steady(s, carry):
        for k in range(NRING):
            extract((s - 1) * NRING + k, k)
            enq(s * NRING + k, k)
        return carry

    lax.fori_loop(1, N_STEPS, steady, 0)
    for k in range(NRING):
        extract((N_STEPS - 1) * NRING + k, k)
    for j in range(N_REM):
        enq(N_STEPS * NRING + j, j)
    for j in range(N_REM):
        extract(N_STEPS * NRING + j, j)

    pltpu.sync_copy(out_vT, outT.at[:, pl.ds(base, B_PER_W)])


def kernel(labels, embedding_table):
    tT = embedding_table.T
    mesh = plsc.VectorSubcoreMesh(core_axis_name="c", subcore_axis_name="s")
    run = pl.kernel(
        _embed_body,
        mesh=mesh,
        out_type=jax.ShapeDtypeStruct((HIDDEN, BATCH), jnp.float32),
        scratch_types=[
            pltpu.VMEM((B_PER_W,), jnp.int32),
            pltpu.VMEM((HIDDEN, B_PER_W), jnp.float32),
        ] + [pltpu.VMEM((HIDDEN, TCOL), jnp.float32)] * NRING
          + [pltpu.SemaphoreType.DMA] * NRING,
        compiler_params=pltpu.CompilerParams(
            needs_layout_passes=False, disable_bounds_checks=True),
    )
    outT = run(tT, labels)
    return outT.T


# speedup vs baseline: 1.0194x; 1.0055x over previous
import jax
import jax.numpy as jnp
from jax import lax
from jax.experimental import pallas as pl
from jax.experimental.pallas import tpu as pltpu
from jax.experimental.pallas import tpu_sc as plsc

BATCH = 16384
HIDDEN = 64
NUM_CORES = 2
NUM_SUBCORES = 16
NUM_WORKERS = NUM_CORES * NUM_SUBCORES
B_PER_W = BATCH // NUM_WORKERS
TCOL = 128
NRING = 11
N_STEPS = B_PER_W // NRING
N_REM = B_PER_W - N_STEPS * NRING


def _embed_body(tT, labels_hbm, outT, lab_v, out_vT,
                b0, b1, b2, b3, b4, b5, b6, b7, b8, b9, b10,
                s0, s1, s2, s3, s4, s5, s6, s7, s8, s9, s10):
    wid = lax.axis_index("s") * NUM_CORES + lax.axis_index("c")
    base = pl.multiple_of(wid * B_PER_W, TCOL)
    pltpu.sync_copy(labels_hbm.at[pl.ds(base, B_PER_W)], lab_v)

    bufs = (b0, b1, b2, b3, b4, b5, b6, b7, b8, b9, b10)
    sems = (s0, s1, s2, s3, s4, s5, s6, s7, s8, s9, s10)
    iota = lax.iota(jnp.int32, 16)

    def get_label(i):
        gb = pl.multiple_of((i >> 4) * 16, 16)
        lv = lab_v[pl.ds(gb, 16)]
        return jnp.max(jnp.where(iota == (i & 15), lv, 0))

    def enq(i, k):
        r = get_label(i)
        rb = pl.multiple_of((r >> 7) * TCOL, TCOL)
        pltpu.async_copy(tT.at[:, pl.ds(rb, TCOL)], bufs[k], sems[k])

    def extract(i, k):
        pltpu.make_async_copy(tT.at[:, pl.ds(0, TCOL)], bufs[k],
                              sems[k]).wait()
        r = get_label(i)
        col = jnp.full((16,), r & (TCOL - 1), jnp.int32)
        tv = jnp.full((16,), i, jnp.int32)
        for q in range(HIDDEN // 16):
            cv = iota + q * 16
            vals = plsc.load_gather(bufs[k], [cv, col])
            plsc.store_scatter(out_vT, [cv, tv], vals)

    for k in range(NRING - 1):
        enq(k, k)

    def steady(s, carry):
        for k in range(NRING):
            enq(s * NRING + k - 1, (k - 1) % NRING)
            extract((s - 1) * NRING + k, k)
        return carry

    lax.fori_loop(1, N_STEPS, steady, 0)
    enq(N_STEPS * NRING - 1, NRING - 1)
    for k in range(NRING):
        extract((N_STEPS - 1) * NRING + k, k)
        if k < N_REM:
            enq(N_STEPS * NRING + k, k)
    for k in range(N_REM):
        extract(N_STEPS * NRING + k, k)

    pltpu.sync_copy(out_vT, outT.at[:, pl.ds(base, B_PER_W)])


def kernel(labels, embedding_table):
    tT = embedding_table.T
    mesh = plsc.VectorSubcoreMesh(core_axis_name="c", subcore_axis_name="s")
    run = pl.kernel(
        _embed_body,
        mesh=mesh,
        out_type=jax.ShapeDtypeStruct((HIDDEN, BATCH), jnp.float32),
        scratch_types=[
            pltpu.VMEM((B_PER_W,), jnp.int32),
            pltpu.VMEM((HIDDEN, B_PER_W), jnp.float32),
        ] + [pltpu.VMEM((HIDDEN, TCOL), jnp.float32)] * NRING
          + [pltpu.SemaphoreType.DMA] * NRING,
        compiler_params=pltpu.CompilerParams(
            needs_layout_passes=False, disable_bounds_checks=True),
    )
    outT = run(tT, labels)
    return outT.T
